# baseline (speedup 1.0000x reference)
import functools

import jax
import jax.numpy as jnp
from jax import lax
from jax.experimental import pallas as pl
from jax.experimental.pallas import tpu as pltpu
from jax.experimental.pallas import tpu_sc as plsc

B = 4096
L = 50
MAX_SEQ_LEN = 128
VOCAB = 10
PAD_TOKEN = 9
EMBED_DIM = 128
HEAD = 56
TAIL = MAX_SEQ_LEN - HEAD
NBUF = 4


def kernel(player_idxs, table):
    idx_dtype = player_idxs.dtype
    info = plsc.get_sparse_core_info()
    nc, ns = info.num_cores, info.num_subcores
    nw = nc * ns
    rpw = B // nw

    mesh = plsc.VectorSubcoreMesh(core_axis_name="c", subcore_axis_name="s")

    @functools.partial(
        pl.kernel,
        mesh=mesh,
        out_type=[
            jax.ShapeDtypeStruct((B, MAX_SEQ_LEN), idx_dtype),
            jax.ShapeDtypeStruct((B, MAX_SEQ_LEN, EMBED_DIM), jnp.float32),
        ],
        scratch_types=[
            pltpu.VMEM((rpw * L + 16,), jnp.int32),
            pltpu.VMEM((rpw, MAX_SEQ_LEN), jnp.int32),
            pltpu.VMEM((VOCAB, EMBED_DIM), jnp.float32),
            pltpu.VMEM_SHARED((TAIL, EMBED_DIM), jnp.float32),
            pltpu.VMEM((NBUF, HEAD, EMBED_DIM), jnp.float32),
            pltpu.SemaphoreType.DMA,
            pltpu.SemaphoreType.DMA,
        ],
    )
    def k(player_hbm, table_hbm, idxs_hbm, emb_hbm, raw_v, idx_v, table_t,
          pad_sp, bufs, ssem, tsem):
        wid = lax.axis_index("s") * nc + lax.axis_index("c")
        base = wid * rpw

        pltpu.sync_copy(table_hbm, table_t)

        @pl.when(lax.axis_index("s") == 0)
        def _():
            pltpu.sync_copy(
                table_t.at[pl.ds(PAD_TOKEN, 1)], pad_sp.at[pl.ds(0, 1)]
            )
            sz = 1
            while sz < TAIL:
                n = min(sz, TAIL - sz)
                pltpu.sync_copy(
                    pad_sp.at[pl.ds(0, n)], pad_sp.at[pl.ds(sz, n)]
                )
                sz += n

        plsc.subcore_barrier()
        pltpu.sync_copy(
            player_hbm.at[pl.ds(base * L, rpw * L)], raw_v.at[pl.ds(0, rpw * L)]
        )

        pad_vec = jnp.full((16,), PAD_TOKEN, jnp.int32)
        col = lax.iota(jnp.int32, 16)
        keep = col < (L - 48)

        def pad_row(r):
            off = r * L
            for cb in range(3):
                idx_v[r, pl.ds(cb * 16, 16)] = raw_v[pl.ds(off + cb * 16, 16)]
            blk = raw_v[pl.ds(off + 48, 16)]
            idx_v[r, pl.ds(48, 16)] = jnp.where(keep, blk, pad_vec)
            for cb in range(4, 8):
                idx_v[r, pl.ds(cb * 16, 16)] = pad_vec

        def build_row(i):
            off = i * L
            slot = i % NBUF
            for q in range(4):
                blk = raw_v[pl.ds(off + q * 16, 16)]
                for lane in range(16):
                    p = q * 16 + lane
                    if p >= L:
                        break
                    t = blk[lane]
                    for cb in range(8):
                        bufs[slot, p, pl.ds(cb * 16, 16)] = table_t[
                            t, pl.ds(cb * 16, 16)
                        ]
            for p in range(L, HEAD):
                for cb in range(8):
                    bufs[slot, p, pl.ds(cb * 16, 16)] = table_t[
                        PAD_TOKEN, pl.ds(cb * 16, 16)
                    ]

        niter = rpw

        def s_desc(i):
            return pltpu.make_async_copy(
                bufs.at[i % NBUF], emb_hbm.at[base + i, pl.ds(0, HEAD)], ssem,
            )

        def t_desc(i):
            return pltpu.make_async_copy(
                pad_sp, emb_hbm.at[base + i, pl.ds(HEAD, TAIL)], tsem,
            )

        def body(i, carry):
            @pl.when(i >= NBUF)
            def _():
                s_desc(i - NBUF).wait()
                t_desc(i - NBUF).wait()

            pad_row(i)
            build_row(i)
            s_desc(i).start()
            t_desc(i).start()
            return carry

        lax.fori_loop(0, niter, body, 0)

        idx_out = pltpu.make_async_copy(
            idx_v, idxs_hbm.at[pl.ds(base, rpw), :], ssem
        )
        idx_out.start()
        for i in range(NBUF, 0, -1):
            s_desc(niter - i).wait()
            t_desc(niter - i).wait()
        idx_out.wait()

    idxs, emb = k(player_idxs.reshape(-1), table)
    return (idxs.astype(idx_dtype), emb)


# speedup vs baseline: 2.1341x; 2.1341x over previous
import functools

import jax
import jax.numpy as jnp
from jax import lax
from jax.experimental import pallas as pl
from jax.experimental.pallas import tpu as pltpu
from jax.experimental.pallas import tpu_sc as plsc

B = 4096
L = 50
MAX_SEQ_LEN = 128
VOCAB = 10
PAD_TOKEN = 9
EMBED_DIM = 128
NBUF = 3
RPS = 2


def kernel(player_idxs, table):
    idx_dtype = player_idxs.dtype
    info = plsc.get_sparse_core_info()
    nc, ns = info.num_cores, info.num_subcores
    nw = nc * ns
    rpw = B // nw

    mesh = plsc.VectorSubcoreMesh(core_axis_name="c", subcore_axis_name="s")

    @functools.partial(
        pl.kernel,
        mesh=mesh,
        out_type=[
            jax.ShapeDtypeStruct((B, MAX_SEQ_LEN), idx_dtype),
            jax.ShapeDtypeStruct((B, MAX_SEQ_LEN, EMBED_DIM), jnp.float32),
        ],
        scratch_types=[
            pltpu.VMEM((rpw * L + 16,), jnp.int32),
            pltpu.VMEM((rpw, MAX_SEQ_LEN), jnp.int32),
            pltpu.VMEM_SHARED((VOCAB, EMBED_DIM), jnp.float32),
            pltpu.VMEM((NBUF, RPS, MAX_SEQ_LEN, EMBED_DIM), jnp.float32),
            pltpu.SemaphoreType.DMA,
            pltpu.SemaphoreType.DMA,
        ],
    )
    def k(player_hbm, table_hbm, idxs_hbm, emb_hbm, raw_v, idx_v, table_v,
          bufs, gsem, ssem):
        wid = lax.axis_index("s") * nc + lax.axis_index("c")
        base = wid * rpw

        @pl.when(lax.axis_index("s") == 0)
        def _():
            pltpu.sync_copy(table_hbm, table_v)

        plsc.subcore_barrier()
        pltpu.sync_copy(
            player_hbm.at[pl.ds(base * L, rpw * L)], raw_v.at[pl.ds(0, rpw * L)]
        )

        pad_vec = jnp.full((16,), PAD_TOKEN, jnp.int32)
        col = lax.iota(jnp.int32, 16)
        keep = col < (L - 48)

        def pad_row(r):
            off = r * L
            for cb in range(3):
                idx_v[r, pl.ds(cb * 16, 16)] = raw_v[pl.ds(off + cb * 16, 16)]
            blk = raw_v[pl.ds(off + 48, 16)]
            idx_v[r, pl.ds(48, 16)] = jnp.where(keep, blk, pad_vec)
            for cb in range(4, 8):
                idx_v[r, pl.ds(cb * 16, 16)] = pad_vec

        niter = rpw // RPS

        def g_desc(i, j):
            return pltpu.make_async_copy(
                table_v.at[idx_v.at[RPS * i + j, pl.ds(0, L)]],
                bufs.at[i % NBUF, j, pl.ds(0, L)],
                gsem,
            )

        def s_desc(i):
            return pltpu.make_async_copy(
                bufs.at[i % NBUF], emb_hbm.at[pl.ds(base + RPS * i, RPS)], ssem,
            )

        pad_row(0)
        tail = MAX_SEQ_LEN - L

        def tail_desc(b, j):
            return pltpu.make_async_copy(
                table_v.at[idx_v.at[0, pl.ds(L, tail)]],
                bufs.at[b, j, pl.ds(L, tail)],
                gsem,
            )

        for b in range(NBUF):
            for j in range(RPS):
                tail_desc(b, j).start()
        for b in range(NBUF):
            for j in range(RPS):
                tail_desc(b, j).wait()

        for i in range(NBUF - 1):
            for j in range(RPS):
                if RPS * i + j > 0:
                    pad_row(RPS * i + j)
                g_desc(i, j).start()

        def body(i, carry):
            @pl.when(i + NBUF - 1 < niter)
            def _():
                for j in range(RPS):
                    pad_row(RPS * (i + NBUF - 1) + j)

                @pl.when(i >= 1)
                def _():
                    s_desc(i - 1).wait()

                for j in range(RPS):
                    g_desc(i + NBUF - 1, j).start()

            for j in range(RPS):
                g_desc(i, j).wait()
            s_desc(i).start()
            return carry

        lax.fori_loop(0, niter, body, 0)

        idx_out = pltpu.make_async_copy(
            idx_v, idxs_hbm.at[pl.ds(base, rpw), :], gsem
        )
        idx_out.start()
        for i in range(NBUF, 0, -1):
            s_desc(niter - i).wait()
        idx_out.wait()

    idxs, emb = k(player_idxs.reshape(-1), table)
    return (idxs.astype(idx_dtype), emb)


# speedup vs baseline: 2.1482x; 1.0066x over previous
import functools

import jax
import jax.numpy as jnp
from jax import lax
from jax.experimental import pallas as pl
from jax.experimental.pallas import tpu as pltpu
from jax.experimental.pallas import tpu_sc as plsc

B = 4096
L = 50
MAX_SEQ_LEN = 128
VOCAB = 10
PAD_TOKEN = 9
EMBED_DIM = 128
NBUF = 3
RPS = 2


def kernel(player_idxs, table):
    idx_dtype = player_idxs.dtype
    info = plsc.get_sparse_core_info()
    nc, ns = info.num_cores, info.num_subcores
    nw = nc * ns
    rpw = B // nw

    mesh = plsc.VectorSubcoreMesh(core_axis_name="c", subcore_axis_name="s")

    @functools.partial(
        pl.kernel,
        mesh=mesh,
        out_type=[
            jax.ShapeDtypeStruct((B, MAX_SEQ_LEN), idx_dtype),
            jax.ShapeDtypeStruct((B, MAX_SEQ_LEN, EMBED_DIM), jnp.float32),
        ],
        scratch_types=[
            pltpu.VMEM((rpw * L + 16,), jnp.int32),
            pltpu.VMEM((rpw, MAX_SEQ_LEN), jnp.int32),
            pltpu.VMEM_SHARED((VOCAB, EMBED_DIM), jnp.float32),
            pltpu.VMEM((NBUF, RPS, MAX_SEQ_LEN, EMBED_DIM), jnp.float32),
            pltpu.SemaphoreType.DMA,
            pltpu.SemaphoreType.DMA,
            pltpu.SemaphoreType.DMA,
        ],
    )
    def k(player_hbm, table_hbm, idxs_hbm, emb_hbm, raw_v, idx_v, table_v,
          bufs, gsem, ssem, isem):
        wid = lax.axis_index("s") * nc + lax.axis_index("c")
        base = wid * rpw

        raw_cp = pltpu.make_async_copy(
            player_hbm.at[pl.ds(base * L, rpw * L)],
            raw_v.at[pl.ds(0, rpw * L)],
            isem,
        )
        raw_cp.start()

        @pl.when(lax.axis_index("s") == 0)
        def _():
            pltpu.sync_copy(table_hbm, table_v)

        plsc.subcore_barrier()
        raw_cp.wait()

        pad_vec = jnp.full((16,), PAD_TOKEN, jnp.int32)
        col = lax.iota(jnp.int32, 16)
        keep = col < (L - 48)

        def pad_row(r):
            off = r * L
            for cb in range(3):
                idx_v[r, pl.ds(cb * 16, 16)] = raw_v[pl.ds(off + cb * 16, 16)]
            blk = raw_v[pl.ds(off + 48, 16)]
            idx_v[r, pl.ds(48, 16)] = jnp.where(keep, blk, pad_vec)
            for cb in range(4, 8):
                idx_v[r, pl.ds(cb * 16, 16)] = pad_vec

        niter = rpw // RPS

        def g_desc(i, j):
            return pltpu.make_async_copy(
                table_v.at[idx_v.at[RPS * i + j, pl.ds(0, L)]],
                bufs.at[i % NBUF, j, pl.ds(0, L)],
                gsem,
            )

        def s_desc(i):
            return pltpu.make_async_copy(
                bufs.at[i % NBUF], emb_hbm.at[pl.ds(base + RPS * i, RPS)], ssem,
            )

        pad_row(0)
        tail = MAX_SEQ_LEN - L

        def tail_desc(b, j):
            return pltpu.make_async_copy(
                table_v.at[idx_v.at[0, pl.ds(L, tail)]],
                bufs.at[b, j, pl.ds(L, tail)],
                ssem,
            )

        for b in range(NBUF):
            for j in range(RPS):
                tail_desc(b, j).start()

        for i in range(NBUF - 1):
            for j in range(RPS):
                if RPS * i + j > 0:
                    pad_row(RPS * i + j)
                g_desc(i, j).start()

        for b in range(NBUF):
            for j in range(RPS):
                tail_desc(b, j).wait()

        def body(i, carry):
            @pl.when(i + NBUF - 1 < niter)
            def _():
                for j in range(RPS):
                    pad_row(RPS * (i + NBUF - 1) + j)

                @pl.when(i >= 1)
                def _():
                    s_desc(i - 1).wait()

                for j in range(RPS):
                    g_desc(i + NBUF - 1, j).start()

            for j in range(RPS):
                g_desc(i, j).wait()
            s_desc(i).start()

            @pl.when(i == niter - NBUF + 1)
            def _():
                pltpu.make_async_copy(
                    idx_v, idxs_hbm.at[pl.ds(base, rpw), :], isem
                ).start()

            return carry

        lax.fori_loop(0, niter, body, 0)

        for i in range(NBUF, 0, -1):
            s_desc(niter - i).wait()
        pltpu.make_async_copy(
            idx_v, idxs_hbm.at[pl.ds(base, rpw), :], isem
        ).wait()

    idxs, emb = k(player_idxs.reshape(-1), table)
    return (idxs.astype(idx_dtype), emb)
